# Initial kernel scaffold; baseline (speedup 1.0000x reference)
import jax
import jax.numpy as jnp
from jax.experimental import pallas as pl


def kernel(x, batch, W_ih, W_hh, b_ih, b_hh, W_post, b_post):
    raise NotImplementedError("write your pallas kernel here")


# speedup vs baseline: 7.2897x; 7.2897x over previous
import functools

import jax
import jax.numpy as jnp
from jax import lax
from jax.experimental import pallas as pl
from jax.experimental.pallas import tpu as pltpu
from jax.experimental.pallas import tpu_sc as plsc

N = 50000
D = 256
B = 512
T = 3
DOUT = 256

NUM_WORKERS = 32
SEGS_PER_W = B // NUM_WORKERS
LANES = 16
DK = D // LANES
CHUNK = 64
NEG = -1e30

_GATHER_DNUMS = lax.GatherDimensionNumbers(
    offset_dims=(), collapsed_slice_dims=(0,), start_index_map=(0,))


def _shuffle(v, perm2d):
    return lax.gather(v, perm2d, _GATHER_DNUMS, (1,),
                      mode=lax.GatherScatterMode.PROMISE_IN_BOUNDS)


def _pool_body(x_hbm, starts_hbm, ends_hbm, q_hbm, r_hbm,
               starts_v, ends_v, qv, xbuf, outv):
    wid = lax.axis_index("s") * 2 + lax.axis_index("c")
    base = wid * SEGS_PER_W
    pltpu.sync_copy(starts_hbm.at[pl.ds(base, SEGS_PER_W)], starts_v)
    pltpu.sync_copy(ends_hbm.at[pl.ds(base, SEGS_PER_W)], ends_v)
    pltpu.sync_copy(q_hbm.at[pl.ds(base, SEGS_PER_W)], qv)

    sv = starts_v[...]
    ev = ends_v[...]
    lane = lax.iota(jnp.int32, LANES)
    perms = [(lane ^ sh)[:, None] for sh in (8, 4, 2, 1)]
    for j in range(SEGS_PER_W):
        s0 = sv[j]
        e0 = ev[j]
        qj = [qv[j, pl.ds(k * LANES, LANES)] for k in range(DK)]

        zeros = jnp.zeros((LANES,), jnp.float32)
        m0 = jnp.full((LANES,), NEG, jnp.float32)
        init = (m0, zeros) + tuple(zeros for _ in range(DK))

        a0 = (s0 // 8) * 8
        n_chunks = jnp.where(e0 > s0, (e0 - a0 + CHUNK - 1) // CHUNK, 0)

        def chunk_body(ci, carry, s0=s0, e0=e0, a0=a0, qj=qj):
            a = a0 + ci * CHUNK
            a_dma = jnp.minimum(a, N - CHUNK)
            pltpu.sync_copy(x_hbm.at[pl.ds(a_dma, CHUNK)], xbuf)
            lower = jnp.maximum(a, s0)

            def row_body(i, rc):
                m_v, s_v = rc[0], rc[1]
                rks = rc[2:]
                xk = [xbuf[i, pl.ds(k * LANES, LANES)] for k in range(DK)]
                acc = xk[0] * qj[0]
                for k in range(1, DK):
                    acc = acc + xk[k] * qj[k]
                for perm in perms:
                    acc = acc + _shuffle(acc, perm)
                e_v = acc
                idx = a_dma + i
                valid = jnp.logical_and(idx >= lower, idx < e0)
                gate = lax.convert_element_type(valid, jnp.float32)
                gate_v = jnp.full((LANES,), gate)
                e_eff = gate_v * e_v + (gate_v - 1.0) * 1e30
                m_new = jnp.maximum(m_v, e_eff)
                scale = jnp.exp(m_v - m_new)
                p = jnp.exp(e_eff - m_new) * gate_v
                s_new = s_v * scale + p
                new_rks = tuple(rks[k] * scale + p * xk[k] for k in range(DK))
                return (m_new, s_new) + new_rks

            return lax.fori_loop(0, CHUNK, row_body, carry)

        fin = lax.fori_loop(0, n_chunks, chunk_body, init)
        inv = 1.0 / (fin[1] + 1e-16)
        for k in range(DK):
            outv[j, pl.ds(k * LANES, LANES)] = fin[2 + k] * inv

    pltpu.sync_copy(outv, r_hbm.at[pl.ds(base, SEGS_PER_W)])


_pool = functools.partial(
    pl.kernel,
    out_type=jax.ShapeDtypeStruct((B, D), jnp.float32),
    mesh=plsc.VectorSubcoreMesh(core_axis_name="c", subcore_axis_name="s"),
    scratch_types=[
        pltpu.VMEM((SEGS_PER_W,), jnp.int32),
        pltpu.VMEM((SEGS_PER_W,), jnp.int32),
        pltpu.VMEM((SEGS_PER_W, D), jnp.float32),
        pltpu.VMEM((CHUNK, D), jnp.float32),
        pltpu.VMEM((SEGS_PER_W, D), jnp.float32),
    ],
)(_pool_body)


def _lstm_body(q_ref, r_ref, h_ref, c_ref, wq_ref, wr_ref, whh_ref,
               bih_ref, bhh_ref, h_out, c_out):
    gates = (
        jnp.dot(q_ref[...], wq_ref[...], preferred_element_type=jnp.float32)
        + jnp.dot(r_ref[...], wr_ref[...], preferred_element_type=jnp.float32)
        + jnp.dot(h_ref[...], whh_ref[...], preferred_element_type=jnp.float32)
        + bih_ref[...] + bhh_ref[...]
    )
    i = jax.nn.sigmoid(gates[:, :D])
    f = jax.nn.sigmoid(gates[:, D:2 * D])
    g = jnp.tanh(gates[:, 2 * D:3 * D])
    o = jax.nn.sigmoid(gates[:, 3 * D:])
    c_new = f * c_ref[...] + i * g
    h_out[...] = o * jnp.tanh(c_new)
    c_out[...] = c_new


_lstm = pl.pallas_call(
    _lstm_body,
    out_shape=(
        jax.ShapeDtypeStruct((B, D), jnp.float32),
        jax.ShapeDtypeStruct((B, D), jnp.float32),
    ),
)


def _post_body(q_ref, r_ref, wp1_ref, wp2_ref, b_ref, o_ref):
    o_ref[...] = jnp.maximum(
        jnp.dot(q_ref[...], wp1_ref[...], preferred_element_type=jnp.float32)
        + jnp.dot(r_ref[...], wp2_ref[...], preferred_element_type=jnp.float32)
        + b_ref[...],
        0.0,
    )


_post = pl.pallas_call(
    _post_body,
    out_shape=jax.ShapeDtypeStruct((B, DOUT), jnp.float32),
)


def kernel(x, batch, W_ih, W_hh, b_ih, b_hh, W_post, b_post):
    x = x.astype(jnp.float32)
    b32 = batch.astype(jnp.int32)
    seg_ids = jnp.arange(B, dtype=jnp.int32)
    starts = jnp.searchsorted(b32, seg_ids, side="left").astype(jnp.int32)
    ends = jnp.searchsorted(b32, seg_ids, side="right").astype(jnp.int32)

    wih_t = W_ih.T
    wq = wih_t[:D]
    wr = wih_t[D:]
    whh_t = W_hh.T
    bih2 = b_ih.reshape(1, 4 * D)
    bhh2 = b_hh.reshape(1, 4 * D)
    wpost_t = W_post.T
    wp1 = wpost_t[:D]
    wp2 = wpost_t[D:]
    bpost2 = b_post.reshape(1, DOUT)

    q = jnp.zeros((B, D), jnp.float32)
    r = jnp.zeros((B, D), jnp.float32)
    h = jnp.zeros((B, D), jnp.float32)
    c = jnp.zeros((B, D), jnp.float32)
    for _ in range(T):
        h, c = _lstm(q, r, h, c, wq, wr, whh_t, bih2, bhh2)
        q = h
        r = _pool(x, starts, ends, q)
    return _post(q, r, wp1, wp2, bpost2)


# speedup vs baseline: 7.5215x; 1.0318x over previous
import functools

import jax
import jax.numpy as jnp
from jax import lax
from jax.experimental import pallas as pl
from jax.experimental.pallas import tpu as pltpu
from jax.experimental.pallas import tpu_sc as plsc

N = 50000
D = 256
B = 512
T = 3
DOUT = 256

NUM_WORKERS = 32
SEGS_PER_W = B // NUM_WORKERS
LANES = 16
DK = D // LANES
CHUNK = 64
NEG = -1e30

_GATHER_DNUMS = lax.GatherDimensionNumbers(
    offset_dims=(), collapsed_slice_dims=(0,), start_index_map=(0,))


def _shuffle(v, perm2d):
    return lax.gather(v, perm2d, _GATHER_DNUMS, (1,),
                      mode=lax.GatherScatterMode.PROMISE_IN_BOUNDS)


def _pool_body(x_hbm, starts_hbm, ends_hbm, q_hbm, r_hbm,
               starts_v, ends_v, qv, xbuf, outv, dsem):
    wid = lax.axis_index("s") * 2 + lax.axis_index("c")
    base = wid * SEGS_PER_W
    pltpu.sync_copy(starts_hbm.at[pl.ds(base, SEGS_PER_W)], starts_v)
    pltpu.sync_copy(ends_hbm.at[pl.ds(base, SEGS_PER_W)], ends_v)
    pltpu.sync_copy(q_hbm.at[pl.ds(base, SEGS_PER_W)], qv)

    sv = starts_v[...]
    ev = ends_v[...]
    lane = lax.iota(jnp.int32, LANES)
    perms = [(lane ^ sh)[:, None] for sh in (8, 4, 2, 1)]
    for j in range(SEGS_PER_W):
        s0 = sv[j]
        e0 = ev[j]
        qj = [qv[j, pl.ds(k * LANES, LANES)] for k in range(DK)]

        zeros = jnp.zeros((LANES,), jnp.float32)
        m0 = jnp.full((LANES,), NEG, jnp.float32)
        init = (m0, zeros) + tuple(zeros for _ in range(DK))

        a0 = (s0 // 8) * 8
        n_chunks = jnp.where(e0 > s0, (e0 - a0 + CHUNK - 1) // CHUNK, 0)

        def dma_start(ci, a0=a0):
            a = a0 + ci * CHUNK
            a_dma = jnp.minimum(a, N - CHUNK)
            buf = ci % 2
            pltpu.make_async_copy(
                x_hbm.at[pl.ds(a_dma, CHUNK)], xbuf.at[buf], dsem.at[buf]
            ).start()

        @pl.when(n_chunks > 0)
        def _():
            dma_start(jnp.int32(0))

        def chunk_body(ci, carry, s0=s0, e0=e0, a0=a0, qj=qj,
                       dma_start=dma_start):
            a = a0 + ci * CHUNK
            a_dma = jnp.minimum(a, N - CHUNK)
            buf = ci % 2

            @pl.when(ci + 1 < n_chunks)
            def _():
                dma_start(ci + 1)

            pltpu.make_async_copy(
                x_hbm.at[pl.ds(a_dma, CHUNK)], xbuf.at[buf], dsem.at[buf]
            ).wait()
            lower = jnp.maximum(a, s0)

            def row_body(i, rc):
                m_v, s_v = rc[0], rc[1]
                rks = rc[2:]
                xk = [xbuf[buf, i, pl.ds(k * LANES, LANES)] for k in range(DK)]
                acc = [xk[k] * qj[k] for k in range(DK)]
                while len(acc) > 1:
                    acc = [acc[k] + acc[k + 1] for k in range(0, len(acc), 2)]
                acc = acc[0]
                for perm in perms:
                    acc = acc + _shuffle(acc, perm)
                e_v = acc
                idx = a_dma + i
                valid = jnp.logical_and(idx >= lower, idx < e0)
                gate = lax.convert_element_type(valid, jnp.float32)
                gate_v = jnp.full((LANES,), gate)
                e_eff = gate_v * e_v + (gate_v - 1.0) * 1e30
                m_new = jnp.maximum(m_v, e_eff)
                scale = jnp.exp(m_v - m_new)
                p = jnp.exp(e_eff - m_new) * gate_v
                s_new = s_v * scale + p
                new_rks = tuple(rks[k] * scale + p * xk[k] for k in range(DK))
                return (m_new, s_new) + new_rks

            return lax.fori_loop(0, CHUNK, row_body, carry)

        fin = lax.fori_loop(0, n_chunks, chunk_body, init)
        inv = 1.0 / (fin[1] + 1e-16)
        for k in range(DK):
            outv[j, pl.ds(k * LANES, LANES)] = fin[2 + k] * inv

    pltpu.sync_copy(outv, r_hbm.at[pl.ds(base, SEGS_PER_W)])


_pool = functools.partial(
    pl.kernel,
    out_type=jax.ShapeDtypeStruct((B, D), jnp.float32),
    mesh=plsc.VectorSubcoreMesh(core_axis_name="c", subcore_axis_name="s"),
    scratch_types=[
        pltpu.VMEM((SEGS_PER_W,), jnp.int32),
        pltpu.VMEM((SEGS_PER_W,), jnp.int32),
        pltpu.VMEM((SEGS_PER_W, D), jnp.float32),
        pltpu.VMEM((2, CHUNK, D), jnp.float32),
        pltpu.VMEM((SEGS_PER_W, D), jnp.float32),
        pltpu.SemaphoreType.DMA((2,)),
    ],
)(_pool_body)


def _lstm_body(q_ref, r_ref, h_ref, c_ref, wq_ref, wr_ref, whh_ref,
               bih_ref, bhh_ref, h_out, c_out):
    gates = (
        jnp.dot(q_ref[...], wq_ref[...], preferred_element_type=jnp.float32)
        + jnp.dot(r_ref[...], wr_ref[...], preferred_element_type=jnp.float32)
        + jnp.dot(h_ref[...], whh_ref[...], preferred_element_type=jnp.float32)
        + bih_ref[...] + bhh_ref[...]
    )
    i = jax.nn.sigmoid(gates[:, :D])
    f = jax.nn.sigmoid(gates[:, D:2 * D])
    g = jnp.tanh(gates[:, 2 * D:3 * D])
    o = jax.nn.sigmoid(gates[:, 3 * D:])
    c_new = f * c_ref[...] + i * g
    h_out[...] = o * jnp.tanh(c_new)
    c_out[...] = c_new


_lstm = pl.pallas_call(
    _lstm_body,
    out_shape=(
        jax.ShapeDtypeStruct((B, D), jnp.float32),
        jax.ShapeDtypeStruct((B, D), jnp.float32),
    ),
)


def _post_body(q_ref, r_ref, wp1_ref, wp2_ref, b_ref, o_ref):
    o_ref[...] = jnp.maximum(
        jnp.dot(q_ref[...], wp1_ref[...], preferred_element_type=jnp.float32)
        + jnp.dot(r_ref[...], wp2_ref[...], preferred_element_type=jnp.float32)
        + b_ref[...],
        0.0,
    )


_post = pl.pallas_call(
    _post_body,
    out_shape=jax.ShapeDtypeStruct((B, DOUT), jnp.float32),
)


def kernel(x, batch, W_ih, W_hh, b_ih, b_hh, W_post, b_post):
    x = x.astype(jnp.float32)
    b32 = batch.astype(jnp.int32)
    seg_ids = jnp.arange(B, dtype=jnp.int32)
    starts = jnp.searchsorted(b32, seg_ids, side="left").astype(jnp.int32)
    ends = jnp.searchsorted(b32, seg_ids, side="right").astype(jnp.int32)

    wih_t = W_ih.T
    wq = wih_t[:D]
    wr = wih_t[D:]
    whh_t = W_hh.T
    bih2 = b_ih.reshape(1, 4 * D)
    bhh2 = b_hh.reshape(1, 4 * D)
    wpost_t = W_post.T
    wp1 = wpost_t[:D]
    wp2 = wpost_t[D:]
    bpost2 = b_post.reshape(1, DOUT)

    q = jnp.zeros((B, D), jnp.float32)
    r = jnp.zeros((B, D), jnp.float32)
    h = jnp.zeros((B, D), jnp.float32)
    c = jnp.zeros((B, D), jnp.float32)
    for _ in range(T):
        h, c = _lstm(q, r, h, c, wq, wr, whh_t, bih2, bhh2)
        q = h
        r = _pool(x, starts, ends, q)
    return _post(q, r, wp1, wp2, bpost2)
